# Initial kernel scaffold; baseline (speedup 1.0000x reference)
import functools

import jax
import jax.numpy as jnp
from jax import lax
from jax.experimental import pallas as pl
from jax.experimental.pallas import tpu as pltpu
from jax.experimental.pallas import tpu_sc as plsc

B, L = 4096, 50
EMBED = 800
LANES = 16
NC, NS = 2, 16
NW = NC * NS
RPW = B // NW
LPAD = 56


def _pool(x_pad, emb):
    mesh = plsc.VectorSubcoreMesh(core_axis_name="c", subcore_axis_name="s")

    @functools.partial(
        pl.kernel,
        out_type=jax.ShapeDtypeStruct((B, EMBED), jnp.float32),
        mesh=mesh,
        scratch_types=[
            pltpu.VMEM((RPW, LPAD), jnp.int32),
            pltpu.VMEM((L, EMBED), jnp.float32),
            pltpu.VMEM((EMBED,), jnp.float32),
            pltpu.SemaphoreType.DMA,
        ],
    )
    def k(x_hbm, emb_hbm, out_hbm, idx_v, gbuf, accv, sem):
        wid = lax.axis_index("s") * NC + lax.axis_index("c")
        base = wid * RPW
        pltpu.sync_copy(x_hbm.at[pl.ds(base, RPW)], idx_v)

        def row(r, carry):
            idx = idx_v.at[r, pl.ds(0, L)]
            pltpu.async_copy(emb_hbm.at[idx], gbuf, sem).wait()

            def dchunk(d, c):
                s = d * LANES
                accs = [jnp.zeros((LANES,), jnp.float32) for _ in range(5)]
                for j in range(L):
                    accs[j % 5] = accs[j % 5] + gbuf[j, pl.ds(s, LANES)]
                tot = (accs[0] + accs[1]) + (accs[2] + accs[3]) + accs[4]
                accv[pl.ds(s, LANES)] = tot * (1.0 / L)
                return c

            lax.fori_loop(0, EMBED // LANES, dchunk, 0)
            pltpu.sync_copy(accv, out_hbm.at[base + r])
            return carry

        lax.fori_loop(0, RPW, row, 0)

    return k(x_pad, emb)


BB = 512


def _mlp_body(p, w1, b1, w2, b2, w3, b3, w4, b4, w5, b5, o):
    h = p[...].astype(jnp.bfloat16)
    h = jnp.maximum(jnp.dot(h, w1[...], preferred_element_type=jnp.float32) + b1[...], 0.0)
    h = jnp.maximum(jnp.dot(h.astype(jnp.bfloat16), w2[...], preferred_element_type=jnp.float32) + b2[...], 0.0)
    h = jnp.maximum(jnp.dot(h.astype(jnp.bfloat16), w3[...], preferred_element_type=jnp.float32) + b3[...], 0.0)
    h = jnp.maximum(jnp.dot(h.astype(jnp.bfloat16), w4[...], preferred_element_type=jnp.float32) + b4[...], 0.0)
    o[...] = jnp.dot(h.astype(jnp.bfloat16), w5[...], preferred_element_type=jnp.float32) + b5[...]


def _mlp(pooled, w1, b1, w2, b2, w3, b3, w4, b4, w5p, b5p):
    full = lambda a: pl.BlockSpec(a.shape, lambda i: (0,) * a.ndim)
    return pl.pallas_call(
        _mlp_body,
        grid=(B // BB,),
        in_specs=[pl.BlockSpec((BB, EMBED), lambda i: (i, 0))]
        + [full(a) for a in (w1, b1, w2, b2, w3, b3, w4, b4, w5p, b5p)],
        out_specs=pl.BlockSpec((BB, 128), lambda i: (i, 0)),
        out_shape=jax.ShapeDtypeStruct((B, 128), jnp.float32),
        compiler_params=pltpu.CompilerParams(dimension_semantics=("arbitrary",)),
    )(pooled, w1, b1, w2, b2, w3, b3, w4, b4, w5p, b5p)


def kernel(x, emb, W1, b1, W2, b2, W3, b3, W4, b4, W5, b5):
    x_pad = jnp.pad(x, ((0, 0), (0, LPAD - L)))
    pooled = _pool(x_pad, emb)

    bf = jnp.bfloat16
    w5p = jnp.pad(W5, ((0, 0), (0, 128 - W5.shape[1])))
    b5p = jnp.pad(b5, ((0, 128 - b5.shape[0]),))
    out = _mlp(
        pooled,
        W1.astype(bf), b1.reshape(1, -1),
        W2.astype(bf), b2.reshape(1, -1),
        W3.astype(bf), b3.reshape(1, -1),
        W4.astype(bf), b4.reshape(1, -1),
        w5p.astype(bf), b5p.reshape(1, -1),
    )
    return out[:, : W5.shape[1]]


# speedup vs baseline: 1.5005x; 1.5005x over previous
import functools

import jax
import jax.numpy as jnp
from jax import lax
from jax.experimental import pallas as pl
from jax.experimental.pallas import tpu as pltpu
from jax.experimental.pallas import tpu_sc as plsc

B, L = 4096, 50
EMBED = 800
LANES = 16
NC, NS = 2, 16
NW = NC * NS
RPW = B // NW
LPAD = 56
GRP = 16


def _pool(x_pad, emb):
    mesh = plsc.VectorSubcoreMesh(
        core_axis_name="c", subcore_axis_name="s", num_cores=NC, num_subcores=NS
    )

    @functools.partial(
        pl.kernel,
        out_type=jax.ShapeDtypeStruct((B, EMBED), jnp.float32),
        mesh=mesh,
        scratch_types=[
            pltpu.VMEM((RPW, LPAD), jnp.int32),
            pltpu.VMEM((LPAD, EMBED), jnp.float32),
            pltpu.VMEM((GRP, EMBED), jnp.float32),
            pltpu.SemaphoreType.DMA,
        ],
        compiler_params=pltpu.CompilerParams(use_tc_tiling_on_sc=False),
    )
    def k(x_hbm, emb_hbm, out_hbm, idx_v, gbuf, obuf, sem):
        wid = lax.axis_index("s") * NC + lax.axis_index("c")
        base = wid * RPW
        pltpu.sync_copy(x_hbm.at[pl.ds(base, RPW)], idx_v)

        def group(g, carry):
            def row(rr, c0):
                r = g * GRP + rr
                idx = idx_v.at[r]
                pltpu.async_copy(emb_hbm.at[idx], gbuf, sem).wait()

                def dchunk(d, c):
                    s = d * LANES
                    accs = [jnp.zeros((LANES,), jnp.float32) for _ in range(5)]
                    for j in range(L):
                        accs[j % 5] = accs[j % 5] + gbuf[j, pl.ds(s, LANES)]
                    tot = (accs[0] + accs[1]) + (accs[2] + accs[3]) + accs[4]
                    obuf[rr, pl.ds(s, LANES)] = tot * (1.0 / L)
                    return c

                lax.fori_loop(0, EMBED // LANES, dchunk, 0)
                return c0

            lax.fori_loop(0, GRP, row, 0)
            pltpu.sync_copy(obuf, out_hbm.at[pl.ds(base + g * GRP, GRP)])
            return carry

        lax.fori_loop(0, RPW // GRP, group, 0)

    return k(x_pad, emb)


BB = 512


def _mlp_body(p, w1, b1, w2, b2, w3, b3, w4, b4, w5, b5, o):
    h = p[...].astype(jnp.bfloat16)
    h = jnp.maximum(jnp.dot(h, w1[...], preferred_element_type=jnp.float32) + b1[...], 0.0)
    h = jnp.maximum(jnp.dot(h.astype(jnp.bfloat16), w2[...], preferred_element_type=jnp.float32) + b2[...], 0.0)
    h = jnp.maximum(jnp.dot(h.astype(jnp.bfloat16), w3[...], preferred_element_type=jnp.float32) + b3[...], 0.0)
    h = jnp.maximum(jnp.dot(h.astype(jnp.bfloat16), w4[...], preferred_element_type=jnp.float32) + b4[...], 0.0)
    o[...] = jnp.dot(h.astype(jnp.bfloat16), w5[...], preferred_element_type=jnp.float32) + b5[...]


def _mlp(pooled, w1, b1, w2, b2, w3, b3, w4, b4, w5p, b5p):
    full = lambda a: pl.BlockSpec(a.shape, lambda i: (0,) * a.ndim)
    return pl.pallas_call(
        _mlp_body,
        grid=(B // BB,),
        in_specs=[pl.BlockSpec((BB, EMBED), lambda i: (i, 0))]
        + [full(a) for a in (w1, b1, w2, b2, w3, b3, w4, b4, w5p, b5p)],
        out_specs=pl.BlockSpec((BB, 128), lambda i: (i, 0)),
        out_shape=jax.ShapeDtypeStruct((B, 128), jnp.float32),
        compiler_params=pltpu.CompilerParams(dimension_semantics=("arbitrary",)),
    )(pooled, w1, b1, w2, b2, w3, b3, w4, b4, w5p, b5p)


def kernel(x, emb, W1, b1, W2, b2, W3, b3, W4, b4, W5, b5):
    x_pad = jnp.pad(x, ((0, 0), (0, LPAD - L)))
    pooled = _pool(x_pad, emb)

    bf = jnp.bfloat16
    w5p = jnp.pad(W5, ((0, 0), (0, 128 - W5.shape[1])))
    b5p = jnp.pad(b5, ((0, 128 - b5.shape[0]),))
    out = _mlp(
        pooled,
        W1.astype(bf), b1.reshape(1, -1),
        W2.astype(bf), b2.reshape(1, -1),
        W3.astype(bf), b3.reshape(1, -1),
        W4.astype(bf), b4.reshape(1, -1),
        w5p.astype(bf), b5p.reshape(1, -1),
    )
    return out[:, : W5.shape[1]]
